# Initial kernel scaffold; baseline (speedup 1.0000x reference)
import jax
import jax.numpy as jnp
from jax.experimental import pallas as pl


def kernel(x, table, W1, b1, W2, b2):
    raise NotImplementedError("write your pallas kernel here")


# speedup vs baseline: 4.0461x; 4.0461x over previous
import functools

import jax
import jax.numpy as jnp
from jax import lax
from jax.experimental import pallas as pl
from jax.experimental.pallas import tpu as pltpu
from jax.experimental.pallas import tpu_sc as plsc

VOCAB = 100000
EMB_DIM = 300
HIDDEN = 128
NUM_CLASSES = 5
BATCH = 4096
SEQ = 200

NC = 2
NS = 16
NW = NC * NS
B_PER_W = BATCH // NW
CHUNK = 100
NCHUNK = SEQ // CHUNK
LANES = 16
ND = HIDDEN // LANES


_PROJ_TM = 2000


def _proj_body(t_ref, w_ref, o_ref):
    o_ref[...] = jnp.dot(t_ref[...], w_ref[...],
                         preferred_element_type=jnp.float32)


def _project_table(table, W1):
    grid = (VOCAB // _PROJ_TM,)
    return pl.pallas_call(
        _proj_body,
        grid=grid,
        in_specs=[
            pl.BlockSpec((_PROJ_TM, EMB_DIM), lambda i: (i, 0)),
            pl.BlockSpec((EMB_DIM, HIDDEN), lambda i: (0, 0)),
        ],
        out_specs=pl.BlockSpec((_PROJ_TM, HIDDEN), lambda i: (i, 0)),
        out_shape=jax.ShapeDtypeStruct((VOCAB, HIDDEN), jnp.float32),
    )(table, W1)


_sc_mesh = plsc.VectorSubcoreMesh(core_axis_name="c", subcore_axis_name="s")


@functools.partial(
    pl.kernel,
    mesh=_sc_mesh,
    out_type=jax.ShapeDtypeStruct((BATCH, HIDDEN), jnp.float32),
    scratch_types=[
        pltpu.VMEM((B_PER_W, NCHUNK, CHUNK), jnp.int32),
        pltpu.VMEM((SEQ, HIDDEN), jnp.float32),
        pltpu.VMEM((B_PER_W, HIDDEN), jnp.float32),
        pltpu.SemaphoreType.DMA,
    ],
)
def _pool_kernel(x_hbm, p_hbm, out_hbm, idx_v, rows_v, pooled_v, sem):
    wid = lax.axis_index("s") * NC + lax.axis_index("c")
    base = wid * B_PER_W
    pltpu.sync_copy(x_hbm.at[pl.ds(base, B_PER_W)], idx_v)

    def item_body(i, carry):
        cps = [
            pltpu.async_copy(
                p_hbm.at[idx_v.at[i, j]],
                rows_v.at[pl.ds(j * CHUNK, CHUNK)],
                sem,
            )
            for j in range(NCHUNK)
        ]
        for cp in cps:
            cp.wait()

        def srow(s, accs):
            return tuple(accs[d] + rows_v[s, pl.ds(d * LANES, LANES)]
                         for d in range(ND))

        zeros = tuple(jnp.zeros((LANES,), jnp.float32) for _ in range(ND))
        accs = lax.fori_loop(0, SEQ, srow, zeros)
        for d in range(ND):
            pooled_v[i, pl.ds(d * LANES, LANES)] = accs[d]
        return carry

    lax.fori_loop(0, B_PER_W, item_body, 0)
    pltpu.sync_copy(pooled_v, out_hbm.at[pl.ds(base, B_PER_W)])


def _mlp_body(p_ref, b1_ref, w2_ref, b2_ref, o_ref):
    h = jnp.maximum(p_ref[...] * (1.0 / SEQ) + b1_ref[...][None, :], 0.0)
    o_ref[...] = jnp.dot(h, w2_ref[...],
                         preferred_element_type=jnp.float32) + b2_ref[...][None, :]


def _mlp_tail(pooled_sum, b1, W2, b2):
    return pl.pallas_call(
        _mlp_body,
        out_shape=jax.ShapeDtypeStruct((BATCH, NUM_CLASSES), jnp.float32),
    )(pooled_sum, b1, W2, b2)


def kernel(x, table, W1, b1, W2, b2):
    proj = _project_table(table, W1)
    x_r = x.reshape(BATCH, NCHUNK, CHUNK)
    pooled_sum = _pool_kernel(x_r, proj)
    return _mlp_tail(pooled_sum, b1, W2, b2)
